# baseline (speedup 1.0000x reference)
import functools

import jax
import jax.numpy as jnp
from jax.experimental import pallas as pl
from jax.experimental.pallas import tpu as pltpu

_E = 16
_K = 64
_R = 768
_A = 16
_ARITY = 3
_TILE = 1024


def _centroid_body(atoms_ref, cw_ref, cl_ref, out_ref):
    ab = jnp.sign(atoms_ref[0])
    acc = jnp.zeros((_K, _R), jnp.float32)
    iota_a = jax.lax.broadcasted_iota(jnp.int32, (_K, _A), 1)
    for b in range(_ARITY):
        lg = cl_ref[0, :, b, :]
        z = lg - jnp.max(lg, axis=-1, keepdims=True)
        e = jnp.exp(z)
        soft = e / jnp.sum(e, axis=-1, keepdims=True)
        m = jnp.max(soft, axis=-1, keepdims=True)
        idx = jnp.min(jnp.where(soft == m, iota_a, _A), axis=-1, keepdims=True)
        onehot = (iota_a == idx).astype(jnp.float32)
        sel = jax.lax.dot_general(
            onehot, ab, (((1,), (0,)), ((), ())),
            preferred_element_type=jnp.float32)
        acc = acc + sel * cw_ref[0, :, b:b + 1]
    cw = jnp.sign(acc)
    out_ref[0] = jnp.sum(cw, axis=0, keepdims=True) * (1.0 / _K)


def _main_body(x_ref, w_ref, cent_ref, out_ref, aux_ref):
    step = pl.program_id(0)
    xt = x_ref[...]
    logits = jax.lax.dot_general(
        xt, w_ref[...], (((1,), (1,)), ((), ())),
        preferred_element_type=jnp.float32)
    z = logits - jnp.max(logits, axis=-1, keepdims=True)
    ez = jnp.exp(z)
    probs = ez / jnp.sum(ez, axis=-1, keepdims=True)

    iota_e = jax.lax.broadcasted_iota(jnp.int32, (_TILE, _E), 1)
    m0 = jnp.max(probs, axis=-1, keepdims=True)
    i0 = jnp.min(jnp.where(probs == m0, iota_e, _E), axis=-1, keepdims=True)
    masked = jnp.where(iota_e == i0, -1.0, probs)
    m1 = jnp.max(masked, axis=-1, keepdims=True)
    i1 = jnp.min(jnp.where(masked == m1, iota_e, _E), axis=-1, keepdims=True)
    inv = 1.0 / (m0 + m1)
    eg = jnp.where(iota_e == i0, m0 * inv, 0.0) + jnp.where(
        iota_e == i1, m1 * inv, 0.0)
    out_ref[...] = jax.lax.dot_general(
        eg, cent_ref[...], (((1,), (0,)), ((), ())),
        preferred_element_type=jnp.float32)

    row0 = jnp.sum(probs, axis=0, keepdims=True)
    row1 = jnp.sum((probs > 0).astype(jnp.float32), axis=0, keepdims=True)
    aux_val = jnp.concatenate([row0, row1], axis=0)

    @pl.when(step == 0)
    def _():
        aux_ref[...] = jnp.zeros_like(aux_ref)

    aux_ref[...] += aux_val


@jax.jit
def kernel(x_latent, W_router, atoms, combo_weights, combo_logits):
    B, S, R = x_latent.shape
    N = B * S
    x2 = x_latent.reshape(N, R)

    centroids = pl.pallas_call(
        _centroid_body,
        grid=(_E,),
        in_specs=[
            pl.BlockSpec((1, _A, _R), lambda e: (e, 0, 0)),
            pl.BlockSpec((1, _K, _ARITY), lambda e: (e, 0, 0)),
            pl.BlockSpec((1, _K, _ARITY, _A), lambda e: (e, 0, 0, 0)),
        ],
        out_specs=pl.BlockSpec((1, 1, _R), lambda e: (e, 0, 0)),
        out_shape=jax.ShapeDtypeStruct((_E, 1, _R), jnp.float32),
    )(atoms, combo_weights, combo_logits)
    centroids = centroids.reshape(_E, _R)

    grid = N // _TILE
    combined, aux = pl.pallas_call(
        _main_body,
        grid=(grid,),
        in_specs=[
            pl.BlockSpec((_TILE, R), lambda i: (i, 0)),
            pl.BlockSpec((_E, R), lambda i: (0, 0)),
            pl.BlockSpec((_E, R), lambda i: (0, 0)),
        ],
        out_specs=[
            pl.BlockSpec((_TILE, R), lambda i: (i, 0)),
            pl.BlockSpec((2, _E), lambda i: (0, 0)),
        ],
        out_shape=[
            jax.ShapeDtypeStruct((N, R), jnp.float32),
            jax.ShapeDtypeStruct((2, _E), jnp.float32),
        ],
        compiler_params=pltpu.CompilerParams(
            dimension_semantics=("arbitrary",)),
    )(x2, W_router, centroids)

    inv_n = 1.0 / N
    aux_loss = _E * jnp.sum((aux[0] * inv_n) * (aux[1] * inv_n))
    return combined.reshape(B, S, R), aux_loss


# speedup vs baseline: 1.1178x; 1.1178x over previous
import functools

import jax
import jax.numpy as jnp
from jax.experimental import pallas as pl
from jax.experimental.pallas import tpu as pltpu

_E = 16
_K = 64
_R = 768
_A = 16
_ARITY = 3
_TILE = 1024


def _centroid_body(atoms_ref, cw_ref, cl_ref, out_ref):
    iota_a = jax.lax.broadcasted_iota(jnp.int32, (_K, _A), 1)
    ab = jnp.sign(atoms_ref[0])
    acc = jnp.zeros((_K, _R), jnp.float32)
    for b in range(_ARITY):
        lg = cl_ref[0, :, b, :]
        z = lg - jnp.max(lg, axis=-1, keepdims=True)
        ez = jnp.exp(z)
        soft = ez / jnp.sum(ez, axis=-1, keepdims=True)
        m = jnp.max(soft, axis=-1, keepdims=True)
        idx = jnp.min(jnp.where(soft == m, iota_a, _A), axis=-1,
                      keepdims=True)
        onehot = (iota_a == idx).astype(jnp.float32)
        sel = jax.lax.dot_general(
            onehot, ab, (((1,), (0,)), ((), ())),
            preferred_element_type=jnp.float32)
        acc = acc + sel * cw_ref[0, :, b:b + 1]
    cw = jnp.sign(acc)
    out_ref[0] = jnp.sum(cw, axis=0, keepdims=True) * (1.0 / _K)


def _main_body(x_ref, w_ref, cent_ref, out_ref, aux_ref):
    step = pl.program_id(0)
    xt = x_ref[...]
    logits = jax.lax.dot_general(
        xt, w_ref[...], (((1,), (1,)), ((), ())),
        preferred_element_type=jnp.float32)
    lt = jnp.transpose(logits)

    z = lt - jnp.max(lt, axis=0, keepdims=True)
    ez = jnp.exp(z)
    probs = ez / jnp.sum(ez, axis=0, keepdims=True)

    iota_e = jax.lax.broadcasted_iota(jnp.int32, (_E, _TILE), 0)
    m0 = jnp.max(probs, axis=0, keepdims=True)
    i0 = jnp.min(jnp.where(probs == m0, iota_e, _E), axis=0, keepdims=True)
    masked = jnp.where(iota_e == i0, -1.0, probs)
    m1 = jnp.max(masked, axis=0, keepdims=True)
    i1 = jnp.min(jnp.where(masked == m1, iota_e, _E), axis=0, keepdims=True)
    inv = 1.0 / (m0 + m1)
    eg = jnp.where(iota_e == i0, m0 * inv, 0.0) + jnp.where(
        iota_e == i1, m1 * inv, 0.0)
    out_ref[...] = jax.lax.dot_general(
        eg.astype(jnp.bfloat16), cent_ref[...], (((0,), (0,)), ((), ())),
        preferred_element_type=jnp.float32)

    col0 = jnp.sum(probs, axis=1, keepdims=True)
    col1 = jnp.sum((probs > 0).astype(jnp.float32), axis=1, keepdims=True)
    aux_val = jnp.concatenate([col0, col1], axis=1)

    @pl.when(step == 0)
    def _():
        aux_ref[...] = jnp.zeros_like(aux_ref)

    aux_ref[...] += aux_val


@jax.jit
def kernel(x_latent, W_router, atoms, combo_weights, combo_logits):
    B, S, R = x_latent.shape
    N = B * S
    x2 = x_latent.reshape(N, R)

    centroids = pl.pallas_call(
        _centroid_body,
        grid=(_E,),
        in_specs=[
            pl.BlockSpec((1, _A, _R), lambda e: (e, 0, 0)),
            pl.BlockSpec((1, _K, _ARITY), lambda e: (e, 0, 0)),
            pl.BlockSpec((1, _K, _ARITY, _A), lambda e: (e, 0, 0, 0)),
        ],
        out_specs=pl.BlockSpec((1, 1, _R), lambda e: (e, 0, 0)),
        out_shape=jax.ShapeDtypeStruct((_E, 1, _R), jnp.float32),
    )(atoms, combo_weights, combo_logits)
    cent_bf = centroids.reshape(_E, _R).astype(jnp.bfloat16)

    grid = N // _TILE
    combined, aux = pl.pallas_call(
        _main_body,
        grid=(grid,),
        in_specs=[
            pl.BlockSpec((_TILE, R), lambda i: (i, 0)),
            pl.BlockSpec((_E, R), lambda i: (0, 0)),
            pl.BlockSpec((_E, R), lambda i: (0, 0)),
        ],
        out_specs=[
            pl.BlockSpec((_TILE, R), lambda i: (i, 0)),
            pl.BlockSpec((_E, 2), lambda i: (0, 0)),
        ],
        out_shape=[
            jax.ShapeDtypeStruct((N, R), jnp.float32),
            jax.ShapeDtypeStruct((_E, 2), jnp.float32),
        ],
        compiler_params=pltpu.CompilerParams(
            dimension_semantics=("arbitrary",)),
    )(x2, W_router, cent_bf)

    inv_n = 1.0 / N
    aux_loss = _E * jnp.sum((aux[:, 0] * inv_n) * (aux[:, 1] * inv_n))
    return combined.reshape(B, S, R), aux_loss
